# baseline (speedup 1.0000x reference)
import jax
import jax.numpy as jnp
from jax import lax
from jax.experimental import pallas as pl

NCLS = 1000
BJ = 512


def _onehot_block(x_ref, o_ref):
    i = pl.program_id(0)
    j = pl.program_id(1)
    xrow = x_ref[pl.ds(i, 1), pl.ds(j * BJ, BJ)]
    cls = lax.broadcasted_iota(jnp.int32, (NCLS, BJ), 0)
    o_ref[0] = (xrow == cls).astype(o_ref.dtype)


def kernel(x):
    out_dtype = jax.dtypes.canonicalize_dtype(jnp.int64)
    n, k = x.shape
    nj = n // BJ
    xt = x.astype(jnp.int32).T
    out = pl.pallas_call(
        _onehot_block,
        grid=(k, nj),
        in_specs=[pl.BlockSpec((k, n), lambda i, j: (0, 0))],
        out_specs=pl.BlockSpec((1, NCLS, BJ), lambda i, j: (i, 0, j)),
        out_shape=jax.ShapeDtypeStruct((k, NCLS, n), out_dtype),
    )(xt)
    return out.transpose(2, 0, 1)


# speedup vs baseline: 1.2250x; 1.2250x over previous
import jax
import jax.numpy as jnp
from jax import lax
from jax.experimental import pallas as pl

NCLS = 1000
BJ = 1024


def _onehot_block(x_ref, o_ref):
    i = pl.program_id(0)
    j = pl.program_id(1)
    xrow = x_ref[pl.ds(i, 1), pl.ds(j * BJ, BJ)]
    cls = lax.broadcasted_iota(jnp.int32, (NCLS, BJ), 0)
    o_ref[0] = (xrow == cls).astype(o_ref.dtype)


def kernel(x):
    out_dtype = jax.dtypes.canonicalize_dtype(jnp.int64)
    n, k = x.shape
    nj = n // BJ
    xt = x.astype(jnp.int32).T
    out = pl.pallas_call(
        _onehot_block,
        grid=(k, nj),
        in_specs=[pl.BlockSpec((k, n), lambda i, j: (0, 0))],
        out_specs=pl.BlockSpec((1, NCLS, BJ), lambda i, j: (i, 0, j)),
        out_shape=jax.ShapeDtypeStruct((k, NCLS, n), out_dtype),
    )(xt)
    return out.transpose(2, 0, 1)


# speedup vs baseline: 1.2312x; 1.0051x over previous
import jax
import jax.numpy as jnp
from jax import lax
from jax.experimental import pallas as pl
from jax.experimental.pallas import tpu as pltpu

NCLS = 1000
BJ = 1024


def _onehot_block(x_ref, o_ref):
    i = pl.program_id(0)
    j = pl.program_id(1)
    xrow = x_ref[pl.ds(i, 1), pl.ds(j * BJ, BJ)]
    cls = lax.broadcasted_iota(jnp.int32, (NCLS, BJ), 0)
    o_ref[0] = (xrow == cls).astype(o_ref.dtype)


def kernel(x):
    out_dtype = jax.dtypes.canonicalize_dtype(jnp.int64)
    n, k = x.shape
    nj = n // BJ
    xt = x.astype(jnp.int32).T
    out = pl.pallas_call(
        _onehot_block,
        grid=(k, nj),
        in_specs=[pl.BlockSpec((k, n), lambda i, j: (0, 0))],
        out_specs=pl.BlockSpec((1, NCLS, BJ), lambda i, j: (i, 0, j)),
        out_shape=jax.ShapeDtypeStruct((k, NCLS, n), out_dtype),
        compiler_params=pltpu.CompilerParams(
            dimension_semantics=("parallel", "parallel")
        ),
    )(xt)
    return out.transpose(2, 0, 1)
